# baseline (speedup 1.0000x reference)
import jax
import jax.numpy as jnp
from jax import lax
from jax.experimental import pallas as pl

B, N, K = 8, 60, 10
NB = 120
AF, BF, H, BIN = 89, 6, 300, 11
DEPTH = 3
NK = N * K
NP = N * N
CH = 600
NCH = NP // CH
TOPK = 20


def _relu(x):
    return jnp.maximum(x, 0.0)


def _mm(a, b):
    return jnp.dot(a, b, preferred_element_type=jnp.float32)


def _hi(a, b):
    return jnp.dot(a, b, precision=lax.Precision.HIGHEST,
                   preferred_element_type=jnp.float32)


def _dgT_hi(a, b):
    return lax.dot_general(a, b, (((0,), (0,)), ((), ())),
                           precision=lax.Precision.HIGHEST,
                           preferred_element_type=jnp.float32)


def _body(fa_ref, fb_ref, anb_ref, bnb_ref, nnb_ref, nat_ref, bin_ref, lbl_ref,
          W_in_ref, b_in_ref, W_msg_ref, b_msg_ref, W_upd_ref, b_upd_ref,
          W_a_ref, W_b_ref, v_att_ref, W_lp_ref, W_gp_ref, W_bp_ref,
          b_s_ref, v_s_ref, ps_ref, tv_ref, ti_ref):
    fa = fa_ref[0]
    fb = fb_ref[0]
    anb = anb_ref[0]
    bnb = bnb_ref[0]
    nnb = nnb_ref[0]
    nat = nat_ref[0]

    rows_n = lax.broadcasted_iota(jnp.int32, (N, 1), 0).astype(jnp.float32)
    atom_mask = (rows_n < nat).astype(jnp.float32)
    rows_nk = lax.broadcasted_iota(jnp.int32, (NK, 1), 0)
    cols_n = lax.broadcasted_iota(jnp.int32, (NK, N), 1)
    cols_nb = lax.broadcasted_iota(jnp.int32, (NK, NB), 1)
    ohA = (anb == cols_n).astype(jnp.float32)
    ohB = (bnb == cols_nb).astype(jnp.float32)
    RK = ((rows_nk // K) == cols_n).astype(jnp.float32)
    kcol = (rows_nk % K).astype(jnp.float32)
    nbs_exp = _hi(RK, nnb)
    nb_mask = (kcol < nbs_exp).astype(jnp.float32)

    b_in = b_in_ref[0][None, :]
    b_msg = b_msg_ref[0][None, :]
    b_upd = b_upd_ref[0][None, :]
    b_s = b_s_ref[0][None, :]

    h = _relu(_mm(fa, W_in_ref[...]) + b_in) * atom_mask
    for _ in range(DEPTH):
        nei_a = _hi(ohA, h)
        nei_b = _hi(ohB, fb)
        cat = jnp.concatenate([nei_a, nei_b], axis=1)
        msg = _relu(_mm(cat, W_msg_ref[...]) + b_msg) * nb_mask
        agg = _dgT_hi(RK, msg)
        h = _relu(_mm(jnp.concatenate([h, agg], axis=1), W_upd_ref[...])
                  + b_upd) * atom_mask

    a1 = _mm(h, W_a_ref[...])
    v_att = v_att_ref[...]
    ctx = jnp.zeros((N, H), jnp.float32)
    RTs, Rs, Ts = [], [], []
    for c in range(NCH):
        rows = lax.broadcasted_iota(jnp.int32, (CH, 1), 0) + (c * CH)
        colsc = lax.broadcasted_iota(jnp.int32, (CH, N), 1)
        Rc = ((rows // N) == colsc).astype(jnp.float32)
        Tc = ((rows % N) == colsc).astype(jnp.float32)
        Rs.append(Rc); Ts.append(Tc); RTs.append(Rc + Tc)
        Bb = _mm(bin_ref[0, c * CH:(c + 1) * CH, :], W_b_ref[...])
        pre = _hi(RTs[c], a1) + Bb
        att = jax.nn.sigmoid(_mm(_relu(pre), v_att))
        hj = _hi(Tc, h)
        ctx = ctx + _dgT_hi(Rc, att * hj)

    v_s = v_s_ref[...]
    ps_rows = []
    for c in range(NCH):
        lp = _hi(RTs[c], h)
        gp = _hi(RTs[c], ctx)
        Bp = _mm(bin_ref[0, c * CH:(c + 1) * CH, :], W_bp_ref[...])
        ph = _relu((_mm(lp, W_lp_ref[...]) + _mm(gp, W_gp_ref[...])) + Bp + b_s)
        psc = _mm(ph, v_s)
        ps_rows.append(jnp.transpose(psc))
    ps_mat = jnp.concatenate(ps_rows, axis=0)
    ps_ref[...] = ps_mat[None]

    lbl = lbl_ref[0]
    masked = jnp.where(lbl == -1.0, ps_mat - 10000.0, ps_mat)
    iota_p = (lax.broadcasted_iota(jnp.int32, (NCH, CH), 0) * CH
              + lax.broadcasted_iota(jnp.int32, (NCH, CH), 1))
    iota_k = lax.broadcasted_iota(jnp.int32, (1, TOPK), 1)
    vals = jnp.zeros((1, TOPK), jnp.float32)
    idxs = jnp.zeros((1, TOPK), jnp.int32)
    for t in range(TOPK):
        v = jnp.max(masked)
        i_t = jnp.min(jnp.where(masked == v, iota_p, NP))
        vals = jnp.where(iota_k == t, v, vals)
        idxs = jnp.where(iota_k == t, i_t, idxs)
        masked = jnp.where(iota_p == i_t, -1e30, masked)
    tv_ref[...] = vals[None]
    ti_ref[...] = idxs[None]


def kernel(fatoms, fbonds, atom_nb, bond_nb, num_nbs, n_atoms, binary_feats,
           labels, W_in, b_in, W_msg, b_msg, W_upd, b_upd, W_a, W_b, v_att,
           W_lp, W_gp, W_bp, b_s, v_s):
    anb3 = atom_nb.astype(jnp.int32).reshape(B, NK, 1)
    bnb3 = bond_nb.astype(jnp.int32).reshape(B, NK, 1)
    nnb3 = num_nbs.astype(jnp.float32).reshape(B, N, 1)
    nat3 = n_atoms.astype(jnp.float32).reshape(B, 1, 1)
    bin3 = binary_feats.reshape(B, NP, BIN)
    lbl3 = labels.reshape(B, NCH, CH)
    b_in2 = b_in.reshape(1, H)
    b_msg2 = b_msg.reshape(1, H)
    b_upd2 = b_upd.reshape(1, H)
    b_s2 = b_s.reshape(1, H)

    def bmap(*shape):
        return pl.BlockSpec((1,) + shape, lambda b: (b,) + (0,) * len(shape))

    def wmap(*shape):
        return pl.BlockSpec(shape, lambda b: (0,) * len(shape))

    ps, tv, ti = pl.pallas_call(
        _body,
        grid=(B,),
        in_specs=[
            bmap(N, AF), bmap(NB, BF), bmap(NK, 1), bmap(NK, 1), bmap(N, 1),
            bmap(1, 1), bmap(NP, BIN), bmap(NCH, CH),
            wmap(AF, H), wmap(1, H), wmap(H + BF, H), wmap(1, H),
            wmap(2 * H, H), wmap(1, H), wmap(H, H), wmap(BIN, H),
            wmap(H, 1), wmap(H, H), wmap(H, H), wmap(BIN, H), wmap(1, H),
            wmap(H, 1),
        ],
        out_specs=[
            pl.BlockSpec((1, NCH, CH), lambda b: (b, 0, 0)),
            pl.BlockSpec((1, 1, TOPK), lambda b: (b, 0, 0)),
            pl.BlockSpec((1, 1, TOPK), lambda b: (b, 0, 0)),
        ],
        out_shape=[
            jax.ShapeDtypeStruct((B, NCH, CH), jnp.float32),
            jax.ShapeDtypeStruct((B, 1, TOPK), jnp.float32),
            jax.ShapeDtypeStruct((B, 1, TOPK), jnp.int32),
        ],
    )(fatoms, fbonds, anb3, bnb3, nnb3, nat3, bin3, lbl3,
      W_in, b_in2, W_msg, b_msg2, W_upd, b_upd2, W_a, W_b, v_att,
      W_lp, W_gp, W_bp, b_s2, v_s)
    return (ps.reshape(B, NP), tv[:, 0, :], ti[:, 0, :])


# speedup vs baseline: 2.4302x; 2.4302x over previous
import jax
import jax.numpy as jnp
from jax import lax
from jax.experimental import pallas as pl

B, N, K = 8, 60, 10
NB = 120
AF, BF, H, BIN = 89, 6, 300, 11
DEPTH = 3
NK = N * K
NP = N * N
CH = 600
NCH = NP // CH
TOPK = 20


def _relu(x):
    return jnp.maximum(x, 0.0)


def _mm(a, b):
    return jnp.dot(a, b, preferred_element_type=jnp.float32)


def _hi(a, b):
    return jnp.dot(a, b, precision=lax.Precision.HIGHEST,
                   preferred_element_type=jnp.float32)


def _dgT_hi(a, b):
    return lax.dot_general(a, b, (((0,), (0,)), ((), ())),
                           precision=lax.Precision.HIGHEST,
                           preferred_element_type=jnp.float32)


def _body(fa_ref, fb_ref, anb_ref, bnb_ref, nnb_ref, nat_ref, bin_ref, lbl_ref,
          W_in_ref, b_in_ref, W_msg_ref, b_msg_ref, W_upd_ref, b_upd_ref,
          W_a_ref, W_b_ref, v_att_ref, W_lp_ref, W_gp_ref, W_bp_ref,
          b_s_ref, v_s_ref, ps_ref, tv_ref, ti_ref):
    fa = fa_ref[0]
    fb = fb_ref[0]
    anb = anb_ref[0]
    bnb = bnb_ref[0]
    nnb = nnb_ref[0]
    nat = nat_ref[0]

    rows_n = lax.broadcasted_iota(jnp.int32, (N, 1), 0).astype(jnp.float32)
    atom_mask = (rows_n < nat).astype(jnp.float32)
    rows_nk = lax.broadcasted_iota(jnp.int32, (NK, 1), 0)
    cols_n = lax.broadcasted_iota(jnp.int32, (NK, N), 1)
    cols_nb = lax.broadcasted_iota(jnp.int32, (NK, NB), 1)
    ohA = (anb == cols_n).astype(jnp.float32)
    ohB = (bnb == cols_nb).astype(jnp.float32)
    RK = ((rows_nk // K) == cols_n).astype(jnp.float32)
    kcol = (rows_nk % K).astype(jnp.float32)
    nbs_exp = _hi(RK, nnb)
    nb_mask = (kcol < nbs_exp).astype(jnp.float32)

    b_in = b_in_ref[0][None, :]
    b_msg = b_msg_ref[0][None, :]
    b_upd = b_upd_ref[0][None, :]
    b_s = b_s_ref[0][None, :]

    h = _relu(_mm(fa, W_in_ref[...]) + b_in) * atom_mask
    for _ in range(DEPTH):
        nei_a = _hi(ohA, h)
        nei_b = _hi(ohB, fb)
        cat = jnp.concatenate([nei_a, nei_b], axis=1)
        msg = _relu(_mm(cat, W_msg_ref[...]) + b_msg) * nb_mask
        agg = _dgT_hi(RK, msg)
        h = _relu(_mm(jnp.concatenate([h, agg], axis=1), W_upd_ref[...])
                  + b_upd) * atom_mask

    IC = CH // N
    a1 = _mm(h, W_a_ref[...])
    v_att = v_att_ref[...]
    ctx = jnp.zeros((N, H), jnp.float32)

    def tile_j(x):
        return jnp.concatenate([x] * IC, axis=0)

    def rep_i(x, c):
        return jnp.concatenate(
            [jnp.broadcast_to(x[c * IC + r:c * IC + r + 1], (N, H))
             for r in range(IC)], axis=0)

    a1j = tile_j(a1)
    hj = tile_j(h)
    Bb_full = _mm(bin_ref[0], W_b_ref[...])
    Rs = []
    for c in range(NCH):
        rows = lax.broadcasted_iota(jnp.int32, (CH, 1), 0) + (c * CH)
        colsc = lax.broadcasted_iota(jnp.int32, (CH, N), 1)
        Rc = ((rows // N) == colsc).astype(jnp.float32)
        Rs.append(Rc)
        pre = (rep_i(a1, c) + a1j) + Bb_full[c * CH:(c + 1) * CH]
        att = jax.nn.sigmoid(_mm(_relu(pre), v_att))
        ctx = ctx + _dgT_hi(Rc, att * hj)

    v_s = v_s_ref[...]
    ctxj = tile_j(ctx)
    Bp_full = _mm(bin_ref[0], W_bp_ref[...])
    ps_rows = []
    for c in range(NCH):
        lp = rep_i(h, c) + hj
        gp = rep_i(ctx, c) + ctxj
        Bp = Bp_full[c * CH:(c + 1) * CH]
        ph = _relu((_mm(lp, W_lp_ref[...]) + _mm(gp, W_gp_ref[...])) + Bp + b_s)
        psc = _mm(ph, v_s)
        ps_rows.append(jnp.transpose(psc))
    ps_mat = jnp.concatenate(ps_rows, axis=0)
    ps_ref[...] = ps_mat[None]

    lbl = lbl_ref[0]
    masked = jnp.where(lbl == -1.0, ps_mat - 10000.0, ps_mat)
    iota_p = (lax.broadcasted_iota(jnp.int32, (NCH, CH), 0) * CH
              + lax.broadcasted_iota(jnp.int32, (NCH, CH), 1))
    iota_k = lax.broadcasted_iota(jnp.int32, (1, TOPK), 1)
    vals = jnp.zeros((1, TOPK), jnp.float32)
    idxs = jnp.zeros((1, TOPK), jnp.int32)
    for t in range(TOPK):
        v = jnp.max(masked)
        i_t = jnp.min(jnp.where(masked == v, iota_p, NP))
        vals = jnp.where(iota_k == t, v, vals)
        idxs = jnp.where(iota_k == t, i_t, idxs)
        masked = jnp.where(iota_p == i_t, -1e30, masked)
    tv_ref[...] = vals[None]
    ti_ref[...] = idxs[None]


def kernel(fatoms, fbonds, atom_nb, bond_nb, num_nbs, n_atoms, binary_feats,
           labels, W_in, b_in, W_msg, b_msg, W_upd, b_upd, W_a, W_b, v_att,
           W_lp, W_gp, W_bp, b_s, v_s):
    anb3 = atom_nb.astype(jnp.int32).reshape(B, NK, 1)
    bnb3 = bond_nb.astype(jnp.int32).reshape(B, NK, 1)
    nnb3 = num_nbs.astype(jnp.float32).reshape(B, N, 1)
    nat3 = n_atoms.astype(jnp.float32).reshape(B, 1, 1)
    bin3 = binary_feats.reshape(B, NP, BIN)
    lbl3 = labels.reshape(B, NCH, CH)
    b_in2 = b_in.reshape(1, H)
    b_msg2 = b_msg.reshape(1, H)
    b_upd2 = b_upd.reshape(1, H)
    b_s2 = b_s.reshape(1, H)

    def bmap(*shape):
        return pl.BlockSpec((1,) + shape, lambda b: (b,) + (0,) * len(shape))

    def wmap(*shape):
        return pl.BlockSpec(shape, lambda b: (0,) * len(shape))

    ps, tv, ti = pl.pallas_call(
        _body,
        grid=(B,),
        in_specs=[
            bmap(N, AF), bmap(NB, BF), bmap(NK, 1), bmap(NK, 1), bmap(N, 1),
            bmap(1, 1), bmap(NP, BIN), bmap(NCH, CH),
            wmap(AF, H), wmap(1, H), wmap(H + BF, H), wmap(1, H),
            wmap(2 * H, H), wmap(1, H), wmap(H, H), wmap(BIN, H),
            wmap(H, 1), wmap(H, H), wmap(H, H), wmap(BIN, H), wmap(1, H),
            wmap(H, 1),
        ],
        out_specs=[
            pl.BlockSpec((1, NCH, CH), lambda b: (b, 0, 0)),
            pl.BlockSpec((1, 1, TOPK), lambda b: (b, 0, 0)),
            pl.BlockSpec((1, 1, TOPK), lambda b: (b, 0, 0)),
        ],
        out_shape=[
            jax.ShapeDtypeStruct((B, NCH, CH), jnp.float32),
            jax.ShapeDtypeStruct((B, 1, TOPK), jnp.float32),
            jax.ShapeDtypeStruct((B, 1, TOPK), jnp.int32),
        ],
    )(fatoms, fbonds, anb3, bnb3, nnb3, nat3, bin3, lbl3,
      W_in, b_in2, W_msg, b_msg2, W_upd, b_upd2, W_a, W_b, v_att,
      W_lp, W_gp, W_bp, b_s2, v_s)
    return (ps.reshape(B, NP), tv[:, 0, :], ti[:, 0, :])
